# Initial kernel scaffold; baseline (speedup 1.0000x reference)
import functools

import jax
import jax.numpy as jnp
from jax import lax
from jax.experimental import pallas as pl
from jax.experimental.pallas import tpu as pltpu
from jax.experimental.pallas import tpu_sc as plsc

NC = 2
NS = 16
NW = NC * NS

N_PAD = 10240
E_PAD = 327680
M_BLK = 1024


def _make_hist(e):
    ew = e // NW
    rows = N_PAD // 128
    mesh = plsc.VectorSubcoreMesh(core_axis_name="c", subcore_axis_name="s")

    @functools.partial(
        pl.kernel,
        out_type=jax.ShapeDtypeStruct((NC, rows, 128), jnp.float32),
        mesh=mesh,
        scratch_types=[
            pltpu.VMEM((ew,), jnp.int32),
            pltpu.VMEM((rows, 128), jnp.float32),
            pltpu.VMEM((rows,), jnp.int32),
            pltpu.VMEM_SHARED((rows, 128), jnp.float32),
        ],
    )
    def hist(dst_hbm, out_hbm, idx_v, hist_v, rid_v, hist_sh):
        c = lax.axis_index("c")
        s = lax.axis_index("s")
        wid = s * NC + c

        def zrow(i, carry):
            hist_v[i // 8, pl.ds((i % 8) * 16, 16)] = jnp.zeros((16,), jnp.float32)
            return carry

        lax.fori_loop(0, rows * 8, zrow, None)

        @pl.when(s == 0)
        def _():
            pltpu.sync_copy(hist_v, hist_sh)

        def rrow(j, carry):
            rid_v[pl.ds(j * 16, 16)] = lax.iota(jnp.int32, 16) + j * 16
            return carry

        lax.fori_loop(0, rows // 16, rrow, None)

        pltpu.sync_copy(dst_hbm.at[pl.ds(wid * ew, ew)], idx_v)

        def hstep(i, carry):
            idx = idx_v[pl.ds(i * 16, 16)]
            cnt, last = plsc.scan_count(idx)
            plsc.addupdate_scatter(
                hist_v,
                [idx >> 7, idx & 127],
                cnt.astype(jnp.float32),
                mask=last,
            )
            return carry

        lax.fori_loop(0, ew // 16, hstep, None)

        plsc.subcore_barrier()
        pltpu.sync_copy(hist_v, hist_sh.at[rid_v], add=True)
        plsc.subcore_barrier()

        @pl.when(s == 0)
        def _():
            pltpu.sync_copy(hist_sh, out_hbm.at[c])

    return hist


def _make_spmm(d):
    ew = E_PAD // NW
    chunks = ew // 128
    rpt = N_PAD // NS
    nv = d // 16
    mesh = plsc.VectorSubcoreMesh(core_axis_name="c", subcore_axis_name="s")

    @functools.partial(
        pl.kernel,
        out_type=jax.ShapeDtypeStruct((NC, N_PAD, d), jnp.float32),
        mesh=mesh,
        scratch_types=[
            pltpu.VMEM((128,), jnp.int32),
            pltpu.VMEM((128,), jnp.int32),
            pltpu.VMEM((128, d), jnp.float32),
            pltpu.VMEM((128, d), jnp.float32),
            pltpu.VMEM_SHARED((N_PAD, d), jnp.float32),
            pltpu.SemaphoreType.DMA,
        ],
    )
    def spmm(hp_hbm, src_hbm, dst_hbm, out_hbm, sidx, didx, rows_v, zero_v,
             acc_sh, sem):
        c = lax.axis_index("c")
        s = lax.axis_index("s")
        wid = s * NC + c

        def zrow(i, carry):
            zero_v[i // nv, pl.ds((i % nv) * 16, 16)] = jnp.zeros((16,), jnp.float32)
            return carry

        lax.fori_loop(0, 128 * nv, zrow, None)

        def zacc(k, carry):
            pltpu.sync_copy(zero_v, acc_sh.at[pl.ds(s * rpt + k * 128, 128)])
            return carry

        lax.fori_loop(0, rpt // 128, zacc, None)
        plsc.subcore_barrier()

        base = wid * ew

        def estep(g, carry):
            off = base + g * 128
            pltpu.sync_copy(src_hbm.at[pl.ds(off, 128)], sidx)
            pltpu.sync_copy(dst_hbm.at[pl.ds(off, 128)], didx)
            pltpu.async_copy(hp_hbm.at[sidx], rows_v, sem).wait()
            pltpu.sync_copy(rows_v, acc_sh.at[didx], add=True)
            return carry

        lax.fori_loop(0, chunks, estep, None)

        plsc.subcore_barrier()
        pltpu.sync_copy(acc_sh.at[pl.ds(s * rpt, rpt)],
                        out_hbm.at[c, pl.ds(s * rpt, rpt)])

    return spmm


def _dinv_body(h_ref, o_ref):
    o_ref[...] = lax.rsqrt(h_ref[0] + h_ref[1] + 1.0)


def _mm1_body(x_ref, w_ref, dinv_ref, o_ref):
    h = jnp.dot(x_ref[...], w_ref[...], preferred_element_type=jnp.float32)
    o_ref[...] = h * dinv_ref[...]


def _mid_body(a_ref, h1p_ref, dinv_ref, b1_ref, w2_ref, o_ref):
    dinv = dinv_ref[...]
    x1 = (a_ref[0] + a_ref[1] + h1p_ref[...]) * dinv + b1_ref[...]
    h2 = jnp.maximum(x1, 0.0)
    o_ref[...] = jnp.dot(h2, w2_ref[...], preferred_element_type=jnp.float32) * dinv


def _fin_body(a_ref, h2p_ref, dinv_ref, b2_ref, o_ref):
    o_ref[...] = (a_ref[0] + a_ref[1] + h2p_ref[...]) * dinv_ref[...] + b2_ref[...]


def _row_block(d):
    return pl.BlockSpec((M_BLK, d), lambda i: (i, 0))


def _full_block(shape):
    return pl.BlockSpec(shape, lambda i: tuple(0 for _ in shape))


def kernel(graph, inputs, W1, b1, W2, b2):
    n, d_in = inputs.shape
    e = graph.shape[1]
    d_hid = W1.shape[1]
    d_out = W2.shape[1]
    grid = N_PAD // M_BLK

    src = graph[0]
    dst = graph[1]
    pad = jnp.full((E_PAD - e,), n, dtype=jnp.int32)
    srcp = jnp.concatenate([src, pad])
    dstp = jnp.concatenate([dst, pad])
    xp = jnp.pad(inputs, ((0, N_PAD - n), (0, 0)))

    hist = _make_hist(e)(dst)

    dinv2d = pl.pallas_call(
        _dinv_body,
        out_shape=jax.ShapeDtypeStruct((N_PAD // 128, 128), jnp.float32),
    )(hist)
    dinv = dinv2d.reshape(N_PAD, 1)

    h1p = pl.pallas_call(
        _mm1_body,
        grid=(grid,),
        in_specs=[
            _row_block(d_in),
            _full_block((d_in, d_hid)),
            _row_block(1),
        ],
        out_specs=_row_block(d_hid),
        out_shape=jax.ShapeDtypeStruct((N_PAD, d_hid), jnp.float32),
    )(xp, W1, dinv)

    acc1 = _make_spmm(d_hid)(h1p, srcp, dstp)

    h2p = pl.pallas_call(
        _mid_body,
        grid=(grid,),
        in_specs=[
            pl.BlockSpec((NC, M_BLK, d_hid), lambda i: (0, i, 0)),
            _row_block(d_hid),
            _row_block(1),
            _full_block((1, d_hid)),
            _full_block((d_hid, d_out)),
        ],
        out_specs=_row_block(d_out),
        out_shape=jax.ShapeDtypeStruct((N_PAD, d_out), jnp.float32),
    )(acc1, h1p, dinv, b1.reshape(1, d_hid), W2)

    acc2 = _make_spmm(d_out)(h2p, srcp, dstp)

    out = pl.pallas_call(
        _fin_body,
        grid=(grid,),
        in_specs=[
            pl.BlockSpec((NC, M_BLK, d_out), lambda i: (0, i, 0)),
            _row_block(d_out),
            _row_block(1),
            _full_block((1, d_out)),
        ],
        out_specs=_row_block(d_out),
        out_shape=jax.ShapeDtypeStruct((N_PAD, d_out), jnp.float32),
    )(acc2, h2p, dinv, b2.reshape(1, d_out))

    return out[:n]


# speedup vs baseline: 6.8983x; 6.8983x over previous
import functools

import jax
import jax.numpy as jnp
from jax import lax
from jax.experimental import pallas as pl
from jax.experimental.pallas import tpu as pltpu
from jax.experimental.pallas import tpu_sc as plsc

NC = 2
NS = 16
NW = NC * NS

N_PAD = 10240
E_PAD = 327680
M_BLK = 1024


def _make_hist():
    ew = E_PAD // NW
    chunks = ew // 128
    npt = N_PAD // NS
    mesh = plsc.VectorSubcoreMesh(core_axis_name="c", subcore_axis_name="s")

    @functools.partial(
        pl.kernel,
        out_type=jax.ShapeDtypeStruct((NC, N_PAD), jnp.float32),
        mesh=mesh,
        scratch_types=[
            pltpu.VMEM((128,), jnp.int32),
            pltpu.VMEM((128,), jnp.float32),
            pltpu.VMEM_SHARED((N_PAD,), jnp.float32),
        ],
    )
    def hist(dst_hbm, out_hbm, didx, ones_v, hist_sh):
        c = lax.axis_index("c")
        s = lax.axis_index("s")
        wid = s * NC + c

        z16 = jnp.zeros((16,), jnp.float32)

        def zfill(i, carry):
            ones_v[pl.ds(i * 16, 16)] = z16
            return carry

        lax.fori_loop(0, 128 // 16, zfill, None)

        def zslice(k, carry):
            pltpu.sync_copy(ones_v, hist_sh.at[pl.ds(s * npt + k * 128, 128)])
            return carry

        lax.fori_loop(0, npt // 128, zslice, None)

        o16 = jnp.ones((16,), jnp.float32)

        def ofill(i, carry):
            ones_v[pl.ds(i * 16, 16)] = o16
            return carry

        lax.fori_loop(0, 128 // 16, ofill, None)
        plsc.subcore_barrier()

        base = wid * ew

        def estep(g, carry):
            pltpu.sync_copy(dst_hbm.at[pl.ds(base + g * 128, 128)], didx)
            pltpu.sync_copy(ones_v, hist_sh.at[didx], add=True)
            return carry

        lax.fori_loop(0, chunks, estep, None)

        plsc.subcore_barrier()
        pltpu.sync_copy(hist_sh.at[pl.ds(s * npt, npt)],
                        out_hbm.at[c, pl.ds(s * npt, npt)])

    return hist


def _make_spmm(d):
    ew = E_PAD // NW
    chunks = ew // 128
    rpt = N_PAD // NS
    nv = d // 16
    mesh = plsc.VectorSubcoreMesh(core_axis_name="c", subcore_axis_name="s")

    @functools.partial(
        pl.kernel,
        out_type=jax.ShapeDtypeStruct((NC, N_PAD, d), jnp.float32),
        mesh=mesh,
        scratch_types=[
            pltpu.VMEM((128,), jnp.int32),
            pltpu.VMEM((128,), jnp.int32),
            pltpu.VMEM((128, d), jnp.float32),
            pltpu.VMEM((128, d), jnp.float32),
            pltpu.VMEM_SHARED((N_PAD, d), jnp.float32),
            pltpu.SemaphoreType.DMA,
        ],
    )
    def spmm(hp_hbm, src_hbm, dst_hbm, out_hbm, sidx, didx, rows_v, zero_v,
             acc_sh, sem):
        c = lax.axis_index("c")
        s = lax.axis_index("s")
        wid = s * NC + c

        def zrow(i, carry):
            zero_v[i // nv, pl.ds((i % nv) * 16, 16)] = jnp.zeros((16,), jnp.float32)
            return carry

        lax.fori_loop(0, 128 * nv, zrow, None)

        def zacc(k, carry):
            pltpu.sync_copy(zero_v, acc_sh.at[pl.ds(s * rpt + k * 128, 128)])
            return carry

        lax.fori_loop(0, rpt // 128, zacc, None)
        plsc.subcore_barrier()

        base = wid * ew

        def estep(g, carry):
            off = base + g * 128
            pltpu.sync_copy(src_hbm.at[pl.ds(off, 128)], sidx)
            pltpu.sync_copy(dst_hbm.at[pl.ds(off, 128)], didx)
            pltpu.async_copy(hp_hbm.at[sidx], rows_v, sem).wait()
            pltpu.sync_copy(rows_v, acc_sh.at[didx], add=True)
            return carry

        lax.fori_loop(0, chunks, estep, None)

        plsc.subcore_barrier()
        pltpu.sync_copy(acc_sh.at[pl.ds(s * rpt, rpt)],
                        out_hbm.at[c, pl.ds(s * rpt, rpt)])

    return spmm


def _dinv_body(h_ref, o_ref):
    o_ref[...] = lax.rsqrt(h_ref[0] + h_ref[1] + 1.0)


def _mm1_body(x_ref, w_ref, dinv_ref, o_ref):
    h = jnp.dot(x_ref[...], w_ref[...], preferred_element_type=jnp.float32)
    o_ref[...] = h * dinv_ref[...]


def _mid_body(a_ref, h1p_ref, dinv_ref, b1_ref, w2_ref, o_ref):
    dinv = dinv_ref[...]
    x1 = (a_ref[0] + a_ref[1] + h1p_ref[...]) * dinv + b1_ref[...]
    h2 = jnp.maximum(x1, 0.0)
    o_ref[...] = jnp.dot(h2, w2_ref[...], preferred_element_type=jnp.float32) * dinv


def _fin_body(a_ref, h2p_ref, dinv_ref, b2_ref, o_ref):
    o_ref[...] = (a_ref[0] + a_ref[1] + h2p_ref[...]) * dinv_ref[...] + b2_ref[...]


def _row_block(d):
    return pl.BlockSpec((M_BLK, d), lambda i: (i, 0))


def _full_block(shape):
    return pl.BlockSpec(shape, lambda i: tuple(0 for _ in shape))


def kernel(graph, inputs, W1, b1, W2, b2):
    n, d_in = inputs.shape
    e = graph.shape[1]
    d_hid = W1.shape[1]
    d_out = W2.shape[1]
    grid = N_PAD // M_BLK

    src = graph[0]
    dst = graph[1]
    pad = jnp.full((E_PAD - e,), n, dtype=jnp.int32)
    srcp = jnp.concatenate([src, pad])
    dstp = jnp.concatenate([dst, pad])
    xp = jnp.pad(inputs, ((0, N_PAD - n), (0, 0)))

    hist = _make_hist()(dstp)
    hist3d = hist.reshape(NC, N_PAD // 128, 128)

    dinv2d = pl.pallas_call(
        _dinv_body,
        out_shape=jax.ShapeDtypeStruct((N_PAD // 128, 128), jnp.float32),
    )(hist3d)
    dinv = dinv2d.reshape(N_PAD, 1)

    h1p = pl.pallas_call(
        _mm1_body,
        grid=(grid,),
        in_specs=[
            _row_block(d_in),
            _full_block((d_in, d_hid)),
            _row_block(1),
        ],
        out_specs=_row_block(d_hid),
        out_shape=jax.ShapeDtypeStruct((N_PAD, d_hid), jnp.float32),
    )(xp, W1, dinv)

    spmm = _make_spmm(d_hid)
    acc1 = spmm(h1p, srcp, dstp)

    d_pad = 128
    W2p = jnp.pad(W2, ((0, 0), (0, d_pad - d_out)))
    b2p = jnp.pad(b2, (0, d_pad - d_out))

    h2p = pl.pallas_call(
        _mid_body,
        grid=(grid,),
        in_specs=[
            pl.BlockSpec((NC, M_BLK, d_hid), lambda i: (0, i, 0)),
            _row_block(d_hid),
            _row_block(1),
            _full_block((1, d_hid)),
            _full_block((d_hid, d_pad)),
        ],
        out_specs=_row_block(d_pad),
        out_shape=jax.ShapeDtypeStruct((N_PAD, d_pad), jnp.float32),
    )(acc1, h1p, dinv, b1.reshape(1, d_hid), W2p)

    acc2 = spmm(h2p, srcp, dstp)

    out = pl.pallas_call(
        _fin_body,
        grid=(grid,),
        in_specs=[
            pl.BlockSpec((NC, M_BLK, d_pad), lambda i: (0, i, 0)),
            _row_block(d_pad),
            _row_block(1),
            _full_block((1, d_pad)),
        ],
        out_specs=_row_block(d_pad),
        out_shape=jax.ShapeDtypeStruct((N_PAD, d_pad), jnp.float32),
    )(acc2, h2p, dinv, b2p.reshape(1, d_pad))

    return out[:n, :d_out]


# speedup vs baseline: 8.3917x; 1.2165x over previous
import functools

import jax
import jax.numpy as jnp
from jax import lax
from jax.experimental import pallas as pl
from jax.experimental.pallas import tpu as pltpu
from jax.experimental.pallas import tpu_sc as plsc

NC = 2
NS = 16
NW = NC * NS

N_PAD = 10240
E_PAD = 327680
M_BLK = 1024


def _make_hist():
    ew = E_PAD // NW
    chunks = ew // 128
    npt = N_PAD // NS
    mesh = plsc.VectorSubcoreMesh(core_axis_name="c", subcore_axis_name="s")

    @functools.partial(
        pl.kernel,
        out_type=jax.ShapeDtypeStruct((NC, N_PAD), jnp.float32),
        mesh=mesh,
        scratch_types=[
            pltpu.VMEM((128,), jnp.int32),
            pltpu.VMEM((128,), jnp.float32),
            pltpu.VMEM_SHARED((N_PAD,), jnp.float32),
        ],
    )
    def hist(dst_hbm, out_hbm, didx, ones_v, hist_sh):
        c = lax.axis_index("c")
        s = lax.axis_index("s")
        wid = s * NC + c

        z16 = jnp.zeros((16,), jnp.float32)

        def zfill(i, carry):
            ones_v[pl.ds(i * 16, 16)] = z16
            return carry

        lax.fori_loop(0, 128 // 16, zfill, None)

        def zslice(k, carry):
            pltpu.sync_copy(ones_v, hist_sh.at[pl.ds(s * npt + k * 128, 128)])
            return carry

        lax.fori_loop(0, npt // 128, zslice, None)

        o16 = jnp.ones((16,), jnp.float32)

        def ofill(i, carry):
            ones_v[pl.ds(i * 16, 16)] = o16
            return carry

        lax.fori_loop(0, 128 // 16, ofill, None)
        plsc.subcore_barrier()

        base = wid * ew

        def estep(g, carry):
            pltpu.sync_copy(dst_hbm.at[pl.ds(base + g * 128, 128)], didx)
            pltpu.sync_copy(ones_v, hist_sh.at[didx], add=True)
            return carry

        lax.fori_loop(0, chunks, estep, None)

        plsc.subcore_barrier()
        pltpu.sync_copy(hist_sh.at[pl.ds(s * npt, npt)],
                        out_hbm.at[c, pl.ds(s * npt, npt)])

    return hist


def _make_spmm(d, ch=128):
    ew = E_PAD // NW
    chunks = ew // ch
    rpt = N_PAD // NS
    nv = d // 16
    mesh = plsc.VectorSubcoreMesh(core_axis_name="c", subcore_axis_name="s")

    @functools.partial(
        pl.kernel,
        out_type=jax.ShapeDtypeStruct((NC, N_PAD, d), jnp.float32),
        mesh=mesh,
        scratch_types=[
            pltpu.VMEM((ch,), jnp.int32),
            pltpu.VMEM((ch,), jnp.int32),
            pltpu.VMEM((ch,), jnp.int32),
            pltpu.VMEM((ch,), jnp.int32),
            pltpu.VMEM((ch, d), jnp.float32),
            pltpu.VMEM((ch, d), jnp.float32),
            pltpu.VMEM_SHARED((N_PAD, d), jnp.float32),
            pltpu.SemaphoreType.DMA,
            pltpu.SemaphoreType.DMA,
        ],
    )
    def spmm(hp_hbm, src_hbm, dst_hbm, out_hbm, sidx0, sidx1, didx0, didx1,
             rows0, rows1, acc_sh, sem0, sem1):
        c = lax.axis_index("c")
        s = lax.axis_index("s")
        wid = s * NC + c
        base = wid * ew

        z16 = jnp.zeros((16,), jnp.float32)

        def zrow(i, carry):
            rows0[i // nv, pl.ds((i % nv) * 16, 16)] = z16
            return carry

        lax.fori_loop(0, ch * nv, zrow, None)

        def zacc(k, carry):
            pltpu.sync_copy(rows0, acc_sh.at[pl.ds(s * rpt + k * ch, ch)])
            return carry

        lax.fori_loop(0, rpt // ch, zacc, None)
        plsc.subcore_barrier()

        def _lidx(g, sbuf, dbuf):
            pltpu.sync_copy(src_hbm.at[pl.ds(base + g * ch, ch)], sbuf)
            pltpu.sync_copy(dst_hbm.at[pl.ds(base + g * ch, ch)], dbuf)

        def _gather(sbuf, rbuf, sem):
            pltpu.async_copy(hp_hbm.at[sbuf], rbuf, sem)

        def _wait(sbuf, rbuf, sem):
            pltpu.make_async_copy(hp_hbm.at[sbuf], rbuf, sem).wait()

        def _scat(dbuf, rbuf):
            pltpu.sync_copy(rbuf, acc_sh.at[dbuf], add=True)

        _lidx(0, sidx0, didx0)
        _gather(sidx0, rows0, sem0)

        def body(j, carry):
            g0 = j * 2
            _lidx(g0 + 1, sidx1, didx1)
            _gather(sidx1, rows1, sem1)
            _wait(sidx0, rows0, sem0)
            _scat(didx0, rows0)

            @pl.when(j + 1 < chunks // 2)
            def _():
                _lidx(g0 + 2, sidx0, didx0)
                _gather(sidx0, rows0, sem0)

            _wait(sidx1, rows1, sem1)
            _scat(didx1, rows1)
            return carry

        lax.fori_loop(0, chunks // 2, body, None)

        plsc.subcore_barrier()
        pltpu.sync_copy(acc_sh.at[pl.ds(s * rpt, rpt)],
                        out_hbm.at[c, pl.ds(s * rpt, rpt)])

    return spmm


def _dinv_body(h_ref, o_ref):
    o_ref[...] = lax.rsqrt(h_ref[0] + h_ref[1] + 1.0)


def _mm1_body(x_ref, w_ref, dinv_ref, o_ref):
    h = jnp.dot(x_ref[...], w_ref[...], preferred_element_type=jnp.float32)
    o_ref[...] = h * dinv_ref[...]


def _mid_body(a_ref, h1p_ref, dinv_ref, b1_ref, w2_ref, o_ref):
    dinv = dinv_ref[...]
    x1 = (a_ref[0] + a_ref[1] + h1p_ref[...]) * dinv + b1_ref[...]
    h2 = jnp.maximum(x1, 0.0)
    o_ref[...] = jnp.dot(h2, w2_ref[...], preferred_element_type=jnp.float32) * dinv


def _fin_body(a_ref, h2p_ref, dinv_ref, b2_ref, o_ref):
    o_ref[...] = (a_ref[0] + a_ref[1] + h2p_ref[...]) * dinv_ref[...] + b2_ref[...]


def _row_block(d):
    return pl.BlockSpec((M_BLK, d), lambda i: (i, 0))


def _full_block(shape):
    return pl.BlockSpec(shape, lambda i: tuple(0 for _ in shape))


def kernel(graph, inputs, W1, b1, W2, b2):
    n, d_in = inputs.shape
    e = graph.shape[1]
    d_hid = W1.shape[1]
    d_out = W2.shape[1]
    grid = N_PAD // M_BLK

    src = graph[0]
    dst = graph[1]
    pad = jnp.full((E_PAD - e,), n, dtype=jnp.int32)
    srcp = jnp.concatenate([src, pad])
    dstp = jnp.concatenate([dst, pad])
    xp = jnp.pad(inputs, ((0, N_PAD - n), (0, 0)))

    hist = _make_hist()(dstp)
    hist3d = hist.reshape(NC, N_PAD // 128, 128)

    dinv2d = pl.pallas_call(
        _dinv_body,
        out_shape=jax.ShapeDtypeStruct((N_PAD // 128, 128), jnp.float32),
    )(hist3d)
    dinv = dinv2d.reshape(N_PAD, 1)

    h1p = pl.pallas_call(
        _mm1_body,
        grid=(grid,),
        in_specs=[
            _row_block(d_in),
            _full_block((d_in, d_hid)),
            _row_block(1),
        ],
        out_specs=_row_block(d_hid),
        out_shape=jax.ShapeDtypeStruct((N_PAD, d_hid), jnp.float32),
    )(xp, W1, dinv)

    acc1 = _make_spmm(d_hid)(h1p, srcp, dstp)

    d_pad = 128
    W2p = jnp.pad(W2, ((0, 0), (0, d_pad - d_out)))
    b2p = jnp.pad(b2, (0, d_pad - d_out))

    h2p = pl.pallas_call(
        _mid_body,
        grid=(grid,),
        in_specs=[
            pl.BlockSpec((NC, M_BLK, d_hid), lambda i: (0, i, 0)),
            _row_block(d_hid),
            _row_block(1),
            _full_block((1, d_hid)),
            _full_block((d_hid, d_pad)),
        ],
        out_specs=_row_block(d_pad),
        out_shape=jax.ShapeDtypeStruct((N_PAD, d_pad), jnp.float32),
    )(acc1, h1p, dinv, b1.reshape(1, d_hid), W2p)

    acc2 = _make_spmm(d_pad)(h2p, srcp, dstp)

    out = pl.pallas_call(
        _fin_body,
        grid=(grid,),
        in_specs=[
            pl.BlockSpec((NC, M_BLK, d_pad), lambda i: (0, i, 0)),
            _row_block(d_pad),
            _row_block(1),
            _full_block((1, d_pad)),
        ],
        out_specs=_row_block(d_pad),
        out_shape=jax.ShapeDtypeStruct((N_PAD, d_pad), jnp.float32),
    )(acc2, h2p, dinv, b2p.reshape(1, d_pad))

    return out[:n, :d_out]


# speedup vs baseline: 8.7338x; 1.0408x over previous
import functools

import jax
import jax.numpy as jnp
from jax import lax
from jax.experimental import pallas as pl
from jax.experimental.pallas import tpu as pltpu
from jax.experimental.pallas import tpu_sc as plsc

NC = 2
NS = 16
NW = NC * NS

N_PAD = 10240
E_PAD = 327680
M_BLK = 1024


def _make_hist(ch=512):
    ew = E_PAD // NW
    chunks = ew // ch
    npt = N_PAD // NS
    mesh = plsc.VectorSubcoreMesh(core_axis_name="c", subcore_axis_name="s")

    @functools.partial(
        pl.kernel,
        out_type=jax.ShapeDtypeStruct((NC, N_PAD), jnp.float32),
        mesh=mesh,
        scratch_types=[
            pltpu.VMEM((ew,), jnp.int32),
            pltpu.VMEM((ch,), jnp.float32),
            pltpu.VMEM_SHARED((N_PAD,), jnp.float32),
            pltpu.SemaphoreType.DMA,
        ],
    )
    def hist(dst_hbm, out_hbm, didx, ones_v, hist_sh, semi):
        c = lax.axis_index("c")
        s = lax.axis_index("s")
        wid = s * NC + c
        base = wid * ew

        pltpu.async_copy(dst_hbm.at[pl.ds(base, ew)], didx, semi)

        z16 = jnp.zeros((16,), jnp.float32)

        def zfill(i, carry):
            ones_v[pl.ds(i * 16, 16)] = z16
            return carry

        lax.fori_loop(0, ch // 16, zfill, None)

        def zslice(k, carry):
            pltpu.sync_copy(ones_v.at[pl.ds(0, 128)],
                            hist_sh.at[pl.ds(s * npt + k * 128, 128)])
            return carry

        lax.fori_loop(0, npt // 128, zslice, None)

        o16 = jnp.ones((16,), jnp.float32)

        def ofill(i, carry):
            ones_v[pl.ds(i * 16, 16)] = o16
            return carry

        lax.fori_loop(0, ch // 16, ofill, None)
        pltpu.make_async_copy(dst_hbm.at[pl.ds(base, ew)], didx, semi).wait()
        plsc.subcore_barrier()

        def estep(g, carry):
            pltpu.sync_copy(ones_v, hist_sh.at[didx.at[pl.ds(g * ch, ch)]],
                            add=True)
            return carry

        lax.fori_loop(0, chunks, estep, None)

        plsc.subcore_barrier()
        pltpu.sync_copy(hist_sh.at[pl.ds(s * npt, npt)],
                        out_hbm.at[c, pl.ds(s * npt, npt)])

    return hist


def _make_spmm(d, ch=80):
    ew = E_PAD // NW
    chunks = ew // ch
    rpt = N_PAD // NS
    nv = d // 16
    mesh = plsc.VectorSubcoreMesh(core_axis_name="c", subcore_axis_name="s")

    @functools.partial(
        pl.kernel,
        out_type=jax.ShapeDtypeStruct((NC, N_PAD, d), jnp.float32),
        mesh=mesh,
        scratch_types=[
            pltpu.VMEM((ew,), jnp.int32),
            pltpu.VMEM((ew,), jnp.int32),
            pltpu.VMEM((ch, d), jnp.float32),
            pltpu.VMEM((ch, d), jnp.float32),
            pltpu.VMEM_SHARED((N_PAD, d), jnp.float32),
            pltpu.SemaphoreType.DMA,
            pltpu.SemaphoreType.DMA,
            pltpu.SemaphoreType.DMA,
        ],
    )
    def spmm(hp_hbm, src_hbm, dst_hbm, out_hbm, sidx, didx,
             rows0, rows1, acc_sh, sem0, sem1, semi):
        c = lax.axis_index("c")
        s = lax.axis_index("s")
        wid = s * NC + c
        base = wid * ew

        pltpu.async_copy(src_hbm.at[pl.ds(base, ew)], sidx, semi)
        pltpu.async_copy(dst_hbm.at[pl.ds(base, ew)], didx, semi)

        z16 = jnp.zeros((16,), jnp.float32)

        def zrow(i, carry):
            rows0[i // nv, pl.ds((i % nv) * 16, 16)] = z16
            return carry

        lax.fori_loop(0, ch * nv, zrow, None)

        def zacc(k, carry):
            pltpu.sync_copy(rows0, acc_sh.at[pl.ds(s * rpt + k * ch, ch)])
            return carry

        lax.fori_loop(0, rpt // ch, zacc, None)

        pltpu.make_async_copy(src_hbm.at[pl.ds(base, ew)], sidx, semi).wait()
        pltpu.make_async_copy(dst_hbm.at[pl.ds(base, ew)], didx, semi).wait()
        plsc.subcore_barrier()

        def _gather(g, rbuf, sem):
            pltpu.async_copy(hp_hbm.at[sidx.at[pl.ds(g * ch, ch)]], rbuf, sem)

        def _wait(g, rbuf, sem):
            pltpu.make_async_copy(
                hp_hbm.at[sidx.at[pl.ds(g * ch, ch)]], rbuf, sem).wait()

        def _scat(g, rbuf):
            pltpu.sync_copy(rbuf, acc_sh.at[didx.at[pl.ds(g * ch, ch)]],
                            add=True)

        _gather(0, rows0, sem0)

        def body(j, carry):
            g0 = j * 2
            _gather(g0 + 1, rows1, sem1)
            _wait(g0, rows0, sem0)
            _scat(g0, rows0)

            @pl.when(j + 1 < chunks // 2)
            def _():
                _gather(g0 + 2, rows0, sem0)

            _wait(g0 + 1, rows1, sem1)
            _scat(g0 + 1, rows1)
            return carry

        lax.fori_loop(0, chunks // 2, body, None)

        plsc.subcore_barrier()
        pltpu.sync_copy(acc_sh.at[pl.ds(s * rpt, rpt)],
                        out_hbm.at[c, pl.ds(s * rpt, rpt)])

    return spmm


def _dinv_body(h_ref, o_ref):
    o_ref[...] = lax.rsqrt(h_ref[0] + h_ref[1] + 1.0)


def _mm1_body(x_ref, w_ref, dinv_ref, o_ref):
    h = jnp.dot(x_ref[...], w_ref[...], preferred_element_type=jnp.float32)
    o_ref[...] = h * dinv_ref[...]


def _mid_body(a_ref, h1p_ref, dinv_ref, b1_ref, w2_ref, o_ref):
    dinv = dinv_ref[...]
    x1 = (a_ref[0] + a_ref[1] + h1p_ref[...]) * dinv + b1_ref[...]
    h2 = jnp.maximum(x1, 0.0)
    o_ref[...] = jnp.dot(h2, w2_ref[...], preferred_element_type=jnp.float32) * dinv


def _fin_body(a_ref, h2p_ref, dinv_ref, b2_ref, o_ref):
    o_ref[...] = (a_ref[0] + a_ref[1] + h2p_ref[...]) * dinv_ref[...] + b2_ref[...]


def _row_block(d):
    return pl.BlockSpec((M_BLK, d), lambda i: (i, 0))


def _full_block(shape):
    return pl.BlockSpec(shape, lambda i: tuple(0 for _ in shape))


def kernel(graph, inputs, W1, b1, W2, b2):
    n, d_in = inputs.shape
    e = graph.shape[1]
    d_hid = W1.shape[1]
    d_out = W2.shape[1]
    grid = N_PAD // M_BLK

    src = graph[0]
    dst = graph[1]
    pad = jnp.full((E_PAD - e,), n, dtype=jnp.int32)
    srcp = jnp.concatenate([src, pad])
    dstp = jnp.concatenate([dst, pad])
    xp = jnp.pad(inputs, ((0, N_PAD - n), (0, 0)))

    hist = _make_hist()(dstp)
    hist3d = hist.reshape(NC, N_PAD // 128, 128)

    dinv2d = pl.pallas_call(
        _dinv_body,
        out_shape=jax.ShapeDtypeStruct((N_PAD // 128, 128), jnp.float32),
    )(hist3d)
    dinv = dinv2d.reshape(N_PAD, 1)

    h1p = pl.pallas_call(
        _mm1_body,
        grid=(grid,),
        in_specs=[
            _row_block(d_in),
            _full_block((d_in, d_hid)),
            _row_block(1),
        ],
        out_specs=_row_block(d_hid),
        out_shape=jax.ShapeDtypeStruct((N_PAD, d_hid), jnp.float32),
    )(xp, W1, dinv)

    acc1 = _make_spmm(d_hid)(h1p, srcp, dstp)

    d_pad = 128
    W2p = jnp.pad(W2, ((0, 0), (0, d_pad - d_out)))
    b2p = jnp.pad(b2, (0, d_pad - d_out))

    h2p = pl.pallas_call(
        _mid_body,
        grid=(grid,),
        in_specs=[
            pl.BlockSpec((NC, M_BLK, d_hid), lambda i: (0, i, 0)),
            _row_block(d_hid),
            _row_block(1),
            _full_block((1, d_hid)),
            _full_block((d_hid, d_pad)),
        ],
        out_specs=_row_block(d_pad),
        out_shape=jax.ShapeDtypeStruct((N_PAD, d_pad), jnp.float32),
    )(acc1, h1p, dinv, b1.reshape(1, d_hid), W2p)

    acc2 = _make_spmm(d_pad)(h2p, srcp, dstp)

    out = pl.pallas_call(
        _fin_body,
        grid=(grid,),
        in_specs=[
            pl.BlockSpec((NC, M_BLK, d_pad), lambda i: (0, i, 0)),
            _row_block(d_pad),
            _row_block(1),
            _full_block((1, d_pad)),
        ],
        out_specs=_row_block(d_pad),
        out_shape=jax.ShapeDtypeStruct((N_PAD, d_pad), jnp.float32),
    )(acc2, h2p, dinv, b2p.reshape(1, d_pad))

    return out[:n, :d_out]
